# Initial kernel scaffold; baseline (speedup 1.0000x reference)
import jax
import jax.numpy as jnp
from jax.experimental import pallas as pl


def kernel(x, edge_index, W, b):
    raise NotImplementedError("write your pallas kernel here")


# speedup vs baseline: 9.3938x; 9.3938x over previous
import functools

import jax
import jax.numpy as jnp
from jax import lax
from jax.experimental import pallas as pl
from jax.experimental.pallas import tpu as pltpu
from jax.experimental.pallas import tpu_sc as plsc

NNODE = 10000
NEDGE = 320000
NFEAT = 128

NC = 2
NS = 16
NW = NC * NS
CHUNK = 128
NCH = 80
EPW = NCH * CHUNK
ACCR = 10112
STRIPE = ACCR // NS

_mesh = plsc.VectorSubcoreMesh(
    core_axis_name="c", subcore_axis_name="s", num_cores=NC, num_subcores=NS
)


@functools.partial(
    pl.kernel,
    out_type=jax.ShapeDtypeStruct((NC, ACCR, 16), jnp.float32),
    mesh=_mesh,
    scratch_types=[
        pltpu.VMEM_SHARED((ACCR, 16), jnp.float32),
        pltpu.VMEM((NCH, 2, CHUNK), jnp.int32),
        pltpu.VMEM((CHUNK, 16), jnp.float32),
    ],
)
def _sc_degree(sd_hbm, ones_hbm, zero_hbm, out_hbm, dacc, sd_v, ones_v):
    c = lax.axis_index("c")
    s = lax.axis_index("s")
    r0 = s * STRIPE
    pltpu.sync_copy(zero_hbm.at[pl.ds(r0, STRIPE)], dacc.at[pl.ds(r0, STRIPE)])
    pltpu.sync_copy(sd_hbm.at[c, s], sd_v)
    pltpu.sync_copy(ones_hbm, ones_v)
    plsc.subcore_barrier()

    @pl.loop(0, NCH)
    def _(j):
        pltpu.sync_copy(ones_v, dacc.at[sd_v.at[j, 1]], add=True)

    plsc.subcore_barrier()
    pltpu.sync_copy(dacc.at[pl.ds(r0, STRIPE)], out_hbm.at[c, pl.ds(r0, STRIPE)])


@functools.partial(
    pl.kernel,
    out_type=jax.ShapeDtypeStruct((NC, ACCR, NFEAT), jnp.float32),
    mesh=_mesh,
    scratch_types=[
        pltpu.VMEM_SHARED((ACCR, NFEAT), jnp.float32),
        [pltpu.VMEM((2, CHUNK), jnp.int32)] * 4,
        [pltpu.VMEM((CHUNK, NFEAT), jnp.float32)] * 2,
        [pltpu.SemaphoreType.DMA] * 2,
        [pltpu.SemaphoreType.DMA] * 4,
    ],
)
def _sc_hop(y_hbm, sd_hbm, zero_hbm, out_hbm, acc, ibs, bufs, sems, isems):
    c = lax.axis_index("c")
    s = lax.axis_index("s")
    r0 = s * STRIPE
    pltpu.sync_copy(zero_hbm.at[pl.ds(r0, STRIPE)], acc.at[pl.ds(r0, STRIPE)])

    pltpu.sync_copy(sd_hbm.at[c, s, 0], ibs[0])
    for q in (1, 2, 3):
        pltpu.async_copy(sd_hbm.at[c, s, q], ibs[q], isems[q])
    plsc.subcore_barrier()
    pltpu.async_copy(y_hbm.at[ibs[0].at[0]], bufs[0], sems[0])

    @pl.loop(0, NCH, step=4)
    def _(g):
        for q in range(4):
            j = g + q
            b = q % 2
            nb = (b + 1) % 2
            nq = (q + 1) % 4

            @pl.when(j + 1 < NCH)
            def _():
                pltpu.make_async_copy(sd_hbm.at[c, s, j + 1], ibs[nq], isems[nq]).wait()
                pltpu.async_copy(y_hbm.at[ibs[nq].at[0]], bufs[nb], sems[nb])

            pltpu.make_async_copy(y_hbm.at[ibs[q].at[0]], bufs[b], sems[b]).wait()
            pltpu.sync_copy(bufs[b], acc.at[ibs[q].at[1]], add=True)

            @pl.when(j + 4 < NCH)
            def _():
                pltpu.async_copy(sd_hbm.at[c, s, j + 4], ibs[q], isems[q])

    plsc.subcore_barrier()
    pltpu.sync_copy(acc.at[pl.ds(r0, STRIPE)], out_hbm.at[c, pl.ds(r0, STRIPE)])


def _deg(d0, d1):
    return 1.0 + d0[:NNODE, 0:1] + d1[:NNODE, 0:1]


def _tc_scale_body(x_ref, d0_ref, d1_ref, y_ref):
    deg = _deg(d0_ref[...], d1_ref[...])
    y_ref[...] = x_ref[...] * lax.rsqrt(deg)


def _tc_combine_body(y_ref, a0_ref, a1_ref, d0_ref, d1_ref, y2_ref):
    deg = _deg(d0_ref[...], d1_ref[...])
    z = y_ref[...] + a0_ref[...][:NNODE] + a1_ref[...][:NNODE]
    y2_ref[...] = z / deg


def _tc_final_body(y2_ref, a0_ref, a1_ref, d0_ref, d1_ref, w_ref, b_ref, o_ref):
    deg = _deg(d0_ref[...], d1_ref[...])
    t = (y2_ref[...] + a0_ref[...][:NNODE] + a1_ref[...][:NNODE]) * lax.rsqrt(deg)
    o_ref[...] = (
        jnp.dot(t, w_ref[...], preferred_element_type=jnp.float32) + b_ref[...]
    )


def kernel(x, edge_index, W, b):
    f32 = jnp.float32
    pad = NW * EPW - NEDGE
    src = jnp.concatenate(
        [edge_index[0], jnp.zeros((pad,), jnp.int32)]
    ).reshape(NC, NS, NCH, CHUNK)
    dst = jnp.concatenate(
        [edge_index[1], NNODE + (jnp.arange(pad, dtype=jnp.int32) % 16)]
    ).reshape(NC, NS, NCH, CHUNK)
    sd = jnp.stack([src, dst], axis=3)

    ones16 = jnp.ones((CHUNK, 16), f32)
    zero16 = jnp.zeros((ACCR, 16), f32)
    zero_big = jnp.zeros((ACCR, NFEAT), f32)

    dparts = _sc_degree(sd, ones16, zero16)
    d0, d1 = dparts[0], dparts[1]

    y = pl.pallas_call(
        _tc_scale_body,
        out_shape=jax.ShapeDtypeStruct((NNODE, NFEAT), f32),
    )(x, d0, d1)

    accs = _sc_hop(y, sd, zero_big)
    y2 = pl.pallas_call(
        _tc_combine_body,
        out_shape=jax.ShapeDtypeStruct((NNODE, NFEAT), f32),
    )(y, accs[0], accs[1], d0, d1)

    accs2 = _sc_hop(y2, sd, zero_big)
    out = pl.pallas_call(
        _tc_final_body,
        out_shape=jax.ShapeDtypeStruct((NNODE, NFEAT), f32),
    )(y2, accs2[0], accs2[1], d0, d1, W, b.reshape(1, NFEAT))
    return out


# speedup vs baseline: 10.5523x; 1.1233x over previous
import functools

import jax
import jax.numpy as jnp
from jax import lax
from jax.experimental import pallas as pl
from jax.experimental.pallas import tpu as pltpu
from jax.experimental.pallas import tpu_sc as plsc

NNODE = 10000
NEDGE = 320000
NFEAT = 128

NC = 2
NS = 16
NW = NC * NS
CHUNK = 128
NCH = 80
EPW = NCH * CHUNK
ACCR = 10112
STRIPE = ACCR // NS

_mesh = plsc.VectorSubcoreMesh(
    core_axis_name="c", subcore_axis_name="s", num_cores=NC, num_subcores=NS
)


@functools.partial(
    pl.kernel,
    out_type=jax.ShapeDtypeStruct((NC, ACCR, NFEAT), jnp.float32),
    mesh=_mesh,
    scratch_types=[
        pltpu.VMEM_SHARED((ACCR, NFEAT), jnp.float32),
        pltpu.VMEM((NCH, CHUNK), jnp.int32),
        pltpu.VMEM((CHUNK, NFEAT), jnp.float32),
    ],
)
def _sc_degree(dst_hbm, ones_hbm, zero_hbm, out_hbm, dacc, dst_v, ones_v):
    c = lax.axis_index("c")
    s = lax.axis_index("s")
    r0 = s * STRIPE
    pltpu.sync_copy(zero_hbm.at[pl.ds(r0, STRIPE)], dacc.at[pl.ds(r0, STRIPE)])
    pltpu.sync_copy(dst_hbm.at[c, s], dst_v)
    pltpu.sync_copy(ones_hbm, ones_v)
    plsc.subcore_barrier()

    @pl.loop(0, NCH)
    def _(j):
        pltpu.sync_copy(ones_v, dacc.at[dst_v.at[j]], add=True)

    plsc.subcore_barrier()
    pltpu.sync_copy(dacc.at[pl.ds(r0, STRIPE)], out_hbm.at[c, pl.ds(r0, STRIPE)])


@functools.partial(
    pl.kernel,
    out_type=jax.ShapeDtypeStruct((NC, ACCR, NFEAT), jnp.float32),
    mesh=_mesh,
    scratch_types=[
        pltpu.VMEM_SHARED((ACCR, NFEAT), jnp.float32),
        [pltpu.VMEM((2, CHUNK), jnp.int32)] * 4,
        [pltpu.VMEM((CHUNK, NFEAT), jnp.float32)] * 2,
        [pltpu.SemaphoreType.DMA] * 2,
        [pltpu.SemaphoreType.DMA] * 4,
    ],
)
def _sc_hop(y_hbm, sd_hbm, zero_hbm, out_hbm, acc, ibs, bufs, sems, isems):
    c = lax.axis_index("c")
    s = lax.axis_index("s")
    r0 = s * STRIPE
    pltpu.sync_copy(zero_hbm.at[pl.ds(r0, STRIPE)], acc.at[pl.ds(r0, STRIPE)])

    pltpu.sync_copy(sd_hbm.at[c, s, 0], ibs[0])
    for q in (1, 2, 3):
        pltpu.async_copy(sd_hbm.at[c, s, q], ibs[q], isems[q])
    plsc.subcore_barrier()
    pltpu.async_copy(y_hbm.at[ibs[0].at[0]], bufs[0], sems[0])

    @pl.loop(0, NCH, step=4)
    def _(g):
        for q in range(4):
            j = g + q
            b = q % 2
            nb = (b + 1) % 2
            nq = (q + 1) % 4

            @pl.when(j + 1 < NCH)
            def _():
                pltpu.make_async_copy(sd_hbm.at[c, s, j + 1], ibs[nq], isems[nq]).wait()
                pltpu.async_copy(y_hbm.at[ibs[nq].at[0]], bufs[nb], sems[nb])

            pltpu.make_async_copy(y_hbm.at[ibs[q].at[0]], bufs[b], sems[b]).wait()
            pltpu.sync_copy(bufs[b], acc.at[ibs[q].at[1]], add=True)

            @pl.when(j + 4 < NCH)
            def _():
                pltpu.async_copy(sd_hbm.at[c, s, j + 4], ibs[q], isems[q])

    plsc.subcore_barrier()
    pltpu.sync_copy(acc.at[pl.ds(r0, STRIPE)], out_hbm.at[c, pl.ds(r0, STRIPE)])


def _deg(d0, d1):
    return 1.0 + d0[:NNODE, 0:1] + d1[:NNODE, 0:1]


def _tc_scale_body(x_ref, d0_ref, d1_ref, y_ref):
    deg = _deg(d0_ref[...], d1_ref[...])
    y_ref[...] = x_ref[...] * lax.rsqrt(deg)


def _tc_combine_body(y_ref, a0_ref, a1_ref, d0_ref, d1_ref, y2_ref):
    deg = _deg(d0_ref[...], d1_ref[...])
    z = y_ref[...] + a0_ref[...][:NNODE] + a1_ref[...][:NNODE]
    y2_ref[...] = z / deg


def _tc_final_body(y2_ref, a0_ref, a1_ref, d0_ref, d1_ref, w_ref, b_ref, o_ref):
    deg = _deg(d0_ref[...], d1_ref[...])
    t = (y2_ref[...] + a0_ref[...][:NNODE] + a1_ref[...][:NNODE]) * lax.rsqrt(deg)
    o_ref[...] = (
        jnp.dot(t, w_ref[...], preferred_element_type=jnp.float32) + b_ref[...]
    )


def kernel(x, edge_index, W, b):
    f32 = jnp.float32
    ppw = EPW - NEDGE // NW
    src = jnp.concatenate(
        [
            edge_index[0].reshape(NW, NEDGE // NW),
            jnp.zeros((NW, ppw), jnp.int32),
        ],
        axis=1,
    ).reshape(NC, NS, NCH, CHUNK)
    dst = jnp.concatenate(
        [
            edge_index[1].reshape(NW, NEDGE // NW),
            NNODE
            + (jnp.arange(NW * ppw, dtype=jnp.int32) % (ACCR - NNODE)).reshape(
                NW, ppw
            ),
        ],
        axis=1,
    ).reshape(NC, NS, NCH, CHUNK)
    sd = jnp.stack([src, dst], axis=3)

    ones_rows = jnp.ones((CHUNK, NFEAT), f32)
    zero_big = jnp.zeros((ACCR, NFEAT), f32)

    dparts = _sc_degree(dst, ones_rows, zero_big)
    d0, d1 = dparts[0], dparts[1]

    y = pl.pallas_call(
        _tc_scale_body,
        out_shape=jax.ShapeDtypeStruct((NNODE, NFEAT), f32),
    )(x, d0, d1)

    accs = _sc_hop(y, sd, zero_big)
    y2 = pl.pallas_call(
        _tc_combine_body,
        out_shape=jax.ShapeDtypeStruct((NNODE, NFEAT), f32),
    )(y, accs[0], accs[1], d0, d1)

    accs2 = _sc_hop(y2, sd, zero_big)
    out = pl.pallas_call(
        _tc_final_body,
        out_shape=jax.ShapeDtypeStruct((NNODE, NFEAT), f32),
    )(y2, accs2[0], accs2[1], d0, d1, W, b.reshape(1, NFEAT))
    return out


# speedup vs baseline: 10.5701x; 1.0017x over previous
import functools

import jax
import jax.numpy as jnp
from jax import lax
from jax.experimental import pallas as pl
from jax.experimental.pallas import tpu as pltpu
from jax.experimental.pallas import tpu_sc as plsc

NNODE = 10000
NEDGE = 320000
NFEAT = 128

NC = 2
NS = 16
NW = NC * NS
CHUNK = 128
NCH = 80
EPW = NCH * CHUNK
ACCR = 10112
STRIPE = ACCR // NS

_mesh = plsc.VectorSubcoreMesh(
    core_axis_name="c", subcore_axis_name="s", num_cores=NC, num_subcores=NS
)


@functools.partial(
    pl.kernel,
    out_type=jax.ShapeDtypeStruct((NC, ACCR, NFEAT), jnp.float32),
    mesh=_mesh,
    scratch_types=[
        pltpu.VMEM_SHARED((ACCR, NFEAT), jnp.float32),
        pltpu.VMEM((NCH, CHUNK), jnp.int32),
        pltpu.VMEM((CHUNK, NFEAT), jnp.float32),
        [pltpu.SemaphoreType.DMA] * 2,
    ],
)
def _sc_degree(dst_hbm, ones_hbm, zero_hbm, out_hbm, dacc, dst_v, ones_v, ssems):
    c = lax.axis_index("c")
    s = lax.axis_index("s")
    r0 = s * STRIPE
    pltpu.sync_copy(zero_hbm.at[pl.ds(r0, STRIPE)], dacc.at[pl.ds(r0, STRIPE)])
    pltpu.sync_copy(dst_hbm.at[c, s], dst_v)
    pltpu.sync_copy(ones_hbm, ones_v)
    plsc.subcore_barrier()

    pltpu.async_copy(ones_v, dacc.at[dst_v.at[0]], ssems[0], add=True)

    @pl.loop(1, NCH)
    def _(j):
        for b in range(2):

            @pl.when(j % 2 == b)
            def _():
                pltpu.async_copy(ones_v, dacc.at[dst_v.at[j]], ssems[b], add=True)
                pltpu.make_async_copy(
                    ones_v, dacc.at[dst_v.at[j - 1]], ssems[1 - b]
                ).wait()

    pltpu.make_async_copy(
        ones_v, dacc.at[dst_v.at[NCH - 1]], ssems[(NCH - 1) % 2]
    ).wait()
    plsc.subcore_barrier()
    pltpu.sync_copy(dacc.at[pl.ds(r0, STRIPE)], out_hbm.at[c, pl.ds(r0, STRIPE)])


@functools.partial(
    pl.kernel,
    out_type=jax.ShapeDtypeStruct((NC, ACCR, NFEAT), jnp.float32),
    mesh=_mesh,
    scratch_types=[
        pltpu.VMEM_SHARED((ACCR, NFEAT), jnp.float32),
        [pltpu.VMEM((2, CHUNK), jnp.int32)] * 4,
        [pltpu.VMEM((CHUNK, NFEAT), jnp.float32)] * 2,
        [pltpu.SemaphoreType.DMA] * 2,
        [pltpu.SemaphoreType.DMA] * 4,
        [pltpu.SemaphoreType.DMA] * 2,
    ],
)
def _sc_hop(y_hbm, sd_hbm, zero_hbm, out_hbm, acc, ibs, bufs, sems, isems, ssems):
    c = lax.axis_index("c")
    s = lax.axis_index("s")
    r0 = s * STRIPE
    pltpu.sync_copy(zero_hbm.at[pl.ds(r0, STRIPE)], acc.at[pl.ds(r0, STRIPE)])

    pltpu.sync_copy(sd_hbm.at[c, s, 0], ibs[0])
    for q in (1, 2):
        pltpu.async_copy(sd_hbm.at[c, s, q], ibs[q], isems[q])
    plsc.subcore_barrier()
    pltpu.async_copy(y_hbm.at[ibs[0].at[0]], bufs[0], sems[0])

    @pl.loop(0, NCH, step=4)
    def _(g):
        for q in range(4):
            j = g + q
            b = q % 2
            nb = (b + 1) % 2
            nq = (q + 1) % 4
            pq = (q + 3) % 4

            @pl.when(j > 0)
            def _():
                pltpu.make_async_copy(
                    bufs[nb], acc.at[ibs[pq].at[1]], ssems[nb]
                ).wait()

            @pl.when(j + 1 < NCH)
            def _():
                pltpu.make_async_copy(sd_hbm.at[c, s, j + 1], ibs[nq], isems[nq]).wait()
                pltpu.async_copy(y_hbm.at[ibs[nq].at[0]], bufs[nb], sems[nb])

            @pl.when(j + 3 < NCH)
            def _():
                pltpu.async_copy(sd_hbm.at[c, s, j + 3], ibs[pq], isems[pq])

            pltpu.make_async_copy(y_hbm.at[ibs[q].at[0]], bufs[b], sems[b]).wait()
            pltpu.async_copy(bufs[b], acc.at[ibs[q].at[1]], ssems[b], add=True)

    pltpu.make_async_copy(
        bufs[(NCH - 1) % 2], acc.at[ibs[(NCH - 1) % 4].at[1]], ssems[(NCH - 1) % 2]
    ).wait()
    plsc.subcore_barrier()
    pltpu.sync_copy(acc.at[pl.ds(r0, STRIPE)], out_hbm.at[c, pl.ds(r0, STRIPE)])


def _deg(d0, d1):
    return 1.0 + d0[:NNODE, 0:1] + d1[:NNODE, 0:1]


def _tc_scale_body(x_ref, d0_ref, d1_ref, y_ref):
    deg = _deg(d0_ref[...], d1_ref[...])
    y_ref[...] = x_ref[...] * lax.rsqrt(deg)


def _tc_combine_body(y_ref, a0_ref, a1_ref, d0_ref, d1_ref, y2_ref):
    deg = _deg(d0_ref[...], d1_ref[...])
    z = y_ref[...] + a0_ref[...][:NNODE] + a1_ref[...][:NNODE]
    y2_ref[...] = z / deg


def _tc_final_body(y2_ref, a0_ref, a1_ref, d0_ref, d1_ref, w_ref, b_ref, o_ref):
    deg = _deg(d0_ref[...], d1_ref[...])
    t = (y2_ref[...] + a0_ref[...][:NNODE] + a1_ref[...][:NNODE]) * lax.rsqrt(deg)
    o_ref[...] = (
        jnp.dot(t, w_ref[...], preferred_element_type=jnp.float32) + b_ref[...]
    )


def kernel(x, edge_index, W, b):
    f32 = jnp.float32
    ppw = EPW - NEDGE // NW
    src = jnp.concatenate(
        [
            edge_index[0].reshape(NW, NEDGE // NW),
            jnp.zeros((NW, ppw), jnp.int32),
        ],
        axis=1,
    ).reshape(NC, NS, NCH, CHUNK)
    dst = jnp.concatenate(
        [
            edge_index[1].reshape(NW, NEDGE // NW),
            NNODE
            + (jnp.arange(NW * ppw, dtype=jnp.int32) % (ACCR - NNODE)).reshape(
                NW, ppw
            ),
        ],
        axis=1,
    ).reshape(NC, NS, NCH, CHUNK)
    sd = jnp.stack([src, dst], axis=3)

    ones_rows = jnp.ones((CHUNK, NFEAT), f32)
    zero_big = jnp.zeros((ACCR, NFEAT), f32)

    dparts = _sc_degree(dst, ones_rows, zero_big)
    d0, d1 = dparts[0], dparts[1]

    y = pl.pallas_call(
        _tc_scale_body,
        out_shape=jax.ShapeDtypeStruct((NNODE, NFEAT), f32),
    )(x, d0, d1)

    accs = _sc_hop(y, sd, zero_big)
    y2 = pl.pallas_call(
        _tc_combine_body,
        out_shape=jax.ShapeDtypeStruct((NNODE, NFEAT), f32),
    )(y, accs[0], accs[1], d0, d1)

    accs2 = _sc_hop(y2, sd, zero_big)
    out = pl.pallas_call(
        _tc_final_body,
        out_shape=jax.ShapeDtypeStruct((NNODE, NFEAT), f32),
    )(y2, accs2[0], accs2[1], d0, d1, W, b.reshape(1, NFEAT))
    return out
